# initial kernel scaffold (unmeasured)
import jax
import jax.numpy as jnp
from jax import lax
from jax.experimental import pallas as pl
from jax.experimental.pallas import tpu as pltpu

Z = 4
N_HOPS = Z - 1
ASSIGN_PAD = 128


def _ring_ids():
    mx = lax.axis_index("x")
    my = lax.axis_index("y")
    mz = lax.axis_index("z")
    right = (mz + 1) % Z
    return mx, my, mz, right


def _ag_ring(aug):
    t, w = aug.shape

    def body(aug_ref, out_ref, send_sems, recv_sems):
        mx, my, mz, right = _ring_ids()
        out_ref[pl.ds(mz * t, t), :] = aug_ref[...]
        for h in range(N_HOPS):
            c_send = (mz - h) % Z
            rdma = pltpu.make_async_remote_copy(
                src_ref=out_ref.at[pl.ds(c_send * t, t), :],
                dst_ref=out_ref.at[pl.ds(c_send * t, t), :],
                send_sem=send_sems.at[h],
                recv_sem=recv_sems.at[h],
                device_id=(mx, my, right),
                device_id_type=pl.DeviceIdType.MESH,
            )
            rdma.start()
            rdma.wait()

    return pl.pallas_call(
        body,
        out_shape=jax.ShapeDtypeStruct((Z * t, w), aug.dtype),
        in_specs=[pl.BlockSpec(memory_space=pltpu.VMEM)],
        out_specs=pl.BlockSpec(memory_space=pltpu.VMEM),
        scratch_shapes=[
            pltpu.SemaphoreType.DMA((N_HOPS,)),
            pltpu.SemaphoreType.DMA((N_HOPS,)),
        ],
        compiler_params=pltpu.CompilerParams(collective_id=0),
    )(aug)


def _moe_compute(full, w1, w2, bt=1024):
    n, w = full.shape
    e_loc, d, f = w1.shape
    grid = (n // bt, e_loc)

    def body(full_ref, w1_ref, w2_ref, out_ref):
        e = pl.program_id(1)
        mz = lax.axis_index("z")
        x_blk = full_ref[:, :d]
        a_blk = full_ref[:, d : d + 1]
        e_val = (mz * e_loc + e).astype(jnp.bfloat16)
        xm = jnp.where(a_blk == e_val, x_blk, jnp.zeros_like(x_blk))
        h = jnp.dot(xm, w1_ref[0], preferred_element_type=jnp.float32)
        h = jnp.maximum(h, 0.0).astype(jnp.bfloat16)
        r = jnp.dot(h, w2_ref[0], preferred_element_type=jnp.float32)
        r = r.astype(jnp.bfloat16)

        @pl.when(e == 0)
        def _():
            out_ref[...] = r

        @pl.when(e != 0)
        def _():
            out_ref[...] = out_ref[...] + r

    return pl.pallas_call(
        body,
        grid=grid,
        in_specs=[
            pl.BlockSpec((bt, w), lambda t, e: (t, 0)),
            pl.BlockSpec((1, d, f), lambda t, e: (e, 0, 0)),
            pl.BlockSpec((1, f, d), lambda t, e: (e, 0, 0)),
        ],
        out_specs=pl.BlockSpec((bt, d), lambda t, e: (t, 0)),
        out_shape=jax.ShapeDtypeStruct((n, d), jnp.bfloat16),
        compiler_params=pltpu.CompilerParams(
            dimension_semantics=("parallel", "arbitrary"),
        ),
    )(full, w1, w2)


def _rs_ring(p):
    n, d = p.shape
    t = n // Z

    def body(p_ref, out_ref, acc_ref, rcv_ref, send_sems, recv_sems):
        mx, my, mz, right = _ring_ids()
        for s in range(N_HOPS):
            c = (mz - 1 - s) % Z
            if s == 0:
                acc_ref[...] = p_ref[pl.ds(c * t, t), :]
            else:
                acc_ref[...] = rcv_ref[s - 1] + p_ref[pl.ds(c * t, t), :]
            rdma = pltpu.make_async_remote_copy(
                src_ref=acc_ref,
                dst_ref=rcv_ref.at[s],
                send_sem=send_sems.at[s],
                recv_sem=recv_sems.at[s],
                device_id=(mx, my, right),
                device_id_type=pl.DeviceIdType.MESH,
            )
            rdma.start()
            rdma.wait()
        out_ref[...] = (
            rcv_ref[N_HOPS - 1] + p_ref[pl.ds(mz * t, t), :]
        ).astype(jnp.float32)

    return pl.pallas_call(
        body,
        out_shape=jax.ShapeDtypeStruct((t, d), jnp.float32),
        in_specs=[pl.BlockSpec(memory_space=pltpu.VMEM)],
        out_specs=pl.BlockSpec(memory_space=pltpu.VMEM),
        scratch_shapes=[
            pltpu.VMEM((t, d), jnp.bfloat16),
            pltpu.VMEM((N_HOPS, t, d), jnp.bfloat16),
            pltpu.SemaphoreType.DMA((N_HOPS,)),
            pltpu.SemaphoreType.DMA((N_HOPS,)),
        ],
        compiler_params=pltpu.CompilerParams(collective_id=1),
    )(p)


def kernel(x, assign, W1, W2):
    tloc, d = x.shape
    xb = x.astype(jnp.bfloat16)
    ab = jnp.broadcast_to(
        assign.astype(jnp.bfloat16)[:, None], (tloc, ASSIGN_PAD)
    )
    aug = jnp.concatenate([xb, ab], axis=1)
    full = _ag_ring(aug)
    partial = _moe_compute(
        full, W1.astype(jnp.bfloat16), W2.astype(jnp.bfloat16)
    )
    return _rs_ring(partial)


# baseline (device time: 672921 ns/iter reference)
import jax
import jax.numpy as jnp
from jax import lax
from jax.experimental import pallas as pl
from jax.experimental.pallas import tpu as pltpu

Z = 4
N_HOPS = Z - 1
ASSIGN_PAD = 128


def _ring_ids():
    mx = lax.axis_index("x")
    my = lax.axis_index("y")
    mz = lax.axis_index("z")
    right = (mz + 1) % Z
    return mx, my, mz, right


def _ag_ring(aug):
    t, w = aug.shape

    def body(aug_ref, out_ref, send_sems, recv_sems):
        mx, my, mz, right = _ring_ids()
        out_ref[pl.ds(mz * t, t), :] = aug_ref[...]
        for h in range(N_HOPS):
            c_send = (mz - h) % Z
            rdma = pltpu.make_async_remote_copy(
                src_ref=out_ref.at[pl.ds(c_send * t, t), :],
                dst_ref=out_ref.at[pl.ds(c_send * t, t), :],
                send_sem=send_sems.at[h],
                recv_sem=recv_sems.at[h],
                device_id=(mx, my, right),
                device_id_type=pl.DeviceIdType.MESH,
            )
            rdma.start()
            rdma.wait()

    return pl.pallas_call(
        body,
        out_shape=jax.ShapeDtypeStruct((Z * t, w), aug.dtype),
        in_specs=[pl.BlockSpec(memory_space=pltpu.VMEM)],
        out_specs=pl.BlockSpec(memory_space=pltpu.VMEM),
        scratch_shapes=[
            pltpu.SemaphoreType.DMA((N_HOPS,)),
            pltpu.SemaphoreType.DMA((N_HOPS,)),
        ],
    )(aug)


def _moe_compute(full, w1, w2, bt=1024):
    n, w = full.shape
    e_loc, d, f = w1.shape
    grid = (n // bt, e_loc)

    def body(full_ref, w1_ref, w2_ref, out_ref):
        e = pl.program_id(1)
        mz = lax.axis_index("z")
        x_blk = full_ref[:, :d]
        a_blk = full_ref[:, d : d + 1]
        e_val = (mz * e_loc + e).astype(jnp.bfloat16)
        xm = jnp.where(a_blk == e_val, x_blk, jnp.zeros_like(x_blk))
        h = jnp.dot(xm, w1_ref[0], preferred_element_type=jnp.float32)
        h = jnp.maximum(h, 0.0).astype(jnp.bfloat16)
        r = jnp.dot(h, w2_ref[0], preferred_element_type=jnp.float32)
        r = r.astype(jnp.bfloat16)

        @pl.when(e == 0)
        def _():
            out_ref[...] = r

        @pl.when(e != 0)
        def _():
            out_ref[...] = out_ref[...] + r

    return pl.pallas_call(
        body,
        grid=grid,
        in_specs=[
            pl.BlockSpec((bt, w), lambda t, e: (t, 0)),
            pl.BlockSpec((1, d, f), lambda t, e: (e, 0, 0)),
            pl.BlockSpec((1, f, d), lambda t, e: (e, 0, 0)),
        ],
        out_specs=pl.BlockSpec((bt, d), lambda t, e: (t, 0)),
        out_shape=jax.ShapeDtypeStruct((n, d), jnp.bfloat16),
        compiler_params=pltpu.CompilerParams(
            dimension_semantics=("parallel", "arbitrary"),
        ),
    )(full, w1, w2)


def _rs_ring(p):
    n, d = p.shape
    t = n // Z

    def body(p_ref, out_ref, acc_ref, rcv_ref, send_sems, recv_sems):
        mx, my, mz, right = _ring_ids()
        for s in range(N_HOPS):
            c = (mz - 1 - s) % Z
            if s == 0:
                acc_ref[...] = p_ref[pl.ds(c * t, t), :]
            else:
                acc_ref[...] = rcv_ref[s - 1] + p_ref[pl.ds(c * t, t), :]
            rdma = pltpu.make_async_remote_copy(
                src_ref=acc_ref,
                dst_ref=rcv_ref.at[s],
                send_sem=send_sems.at[s],
                recv_sem=recv_sems.at[s],
                device_id=(mx, my, right),
                device_id_type=pl.DeviceIdType.MESH,
            )
            rdma.start()
            rdma.wait()
        out_ref[...] = (
            rcv_ref[N_HOPS - 1] + p_ref[pl.ds(mz * t, t), :]
        ).astype(jnp.float32)

    return pl.pallas_call(
        body,
        out_shape=jax.ShapeDtypeStruct((t, d), jnp.float32),
        in_specs=[pl.BlockSpec(memory_space=pltpu.VMEM)],
        out_specs=pl.BlockSpec(memory_space=pltpu.VMEM),
        scratch_shapes=[
            pltpu.VMEM((t, d), jnp.bfloat16),
            pltpu.VMEM((N_HOPS, t, d), jnp.bfloat16),
            pltpu.SemaphoreType.DMA((N_HOPS,)),
            pltpu.SemaphoreType.DMA((N_HOPS,)),
        ],
    )(p)


def kernel(x, assign, W1, W2):
    tloc, d = x.shape
    xb = x.astype(jnp.bfloat16)
    ab = jnp.broadcast_to(
        assign.astype(jnp.bfloat16)[:, None], (tloc, ASSIGN_PAD)
    )
    aug = jnp.concatenate([xb, ab], axis=1)
    full = _ag_ring(aug)
    partial = _moe_compute(
        full, W1.astype(jnp.bfloat16), W2.astype(jnp.bfloat16)
    )
    return _rs_ring(partial)


# device time: 479407 ns/iter; 1.4037x vs baseline; 1.4037x over previous
import jax
import jax.numpy as jnp
from jax import lax
from jax.experimental import pallas as pl
from jax.experimental.pallas import tpu as pltpu

Z = 4
N_HOPS = Z - 1
ASSIGN_PAD = 128
BT = 256


def _fused(aug, w1, w2):
    t_loc, wcols = aug.shape
    e_loc, d, f = w1.shape
    n_tiles = t_loc // BT

    def body(
        aug_ref, w1_hbm, w2_hbm, out_ref,
        chunks, wb1, wb2, p_own, acc, rsrecv,
        ag_send, ag_recv, rs_send, rs_recv, wsems,
    ):
        mx = lax.axis_index("x")
        my = lax.axis_index("y")
        mz = lax.axis_index("z")
        right = (mz + 1) % Z
        dev = (mx, my, right)

        def remote(src, dst, ssem, rsem):
            return pltpu.make_async_remote_copy(
                src_ref=src, dst_ref=dst, send_sem=ssem, recv_sem=rsem,
                device_id=dev, device_id_type=pl.DeviceIdType.MESH,
            )

        def compute_partial(load_src, store, load_dst):
            for e in range(e_loc):
                c1 = pltpu.make_async_copy(w1_hbm.at[e], wb1, wsems.at[0])
                c2 = pltpu.make_async_copy(w2_hbm.at[e], wb2, wsems.at[1])
                c1.start()
                c2.start()
                c1.wait()
                c2.wait()
                e_val = (mz * e_loc + e).astype(jnp.bfloat16)

                def tile(ti, _, e=e, e_val=e_val):
                    rows = pl.ds(ti * BT, BT)
                    blk = load_src(rows)
                    xm = jnp.where(blk[:, d : d + 1] == e_val, blk[:, :d], 0)
                    h = jnp.dot(xm, wb1[...], preferred_element_type=jnp.float32)
                    h = jnp.maximum(h, 0.0).astype(jnp.bfloat16)
                    r = jnp.dot(h, wb2[...], preferred_element_type=jnp.float32)
                    r = r.astype(jnp.bfloat16)
                    if e == 0:
                        store(rows, r)
                    else:
                        store(rows, load_dst(rows) + r)
                    return 0

                lax.fori_loop(0, n_tiles, tile, 0, unroll=False)

        def add_into_acc(a_slot, r_slot):
            def tile(ti, _):
                rows = pl.ds(ti * BT, BT)
                acc[a_slot, rows, :] = acc[a_slot, rows, :] + rsrecv[r_slot, rows, :]
                return 0

            lax.fori_loop(0, n_tiles, tile, 0, unroll=False)

        ag0 = remote(aug_ref, chunks.at[0], ag_send.at[0], ag_recv.at[0])
        ag0.start()
        compute_partial(
            lambda rows: aug_ref[rows, :],
            lambda rows, v: p_own.__setitem__((rows, slice(None)), v),
            lambda rows: p_own[rows, :],
        )
        ag0.wait_recv()

        ag1 = remote(chunks.at[0], chunks.at[1], ag_send.at[1], ag_recv.at[1])
        ag1.start()
        compute_partial(
            lambda rows: chunks[0, rows, :],
            lambda rows, v: acc.__setitem__((0, rows, slice(None)), v),
            lambda rows: acc[0, rows, :],
        )
        rs0 = remote(acc.at[0], rsrecv.at[0], rs_send.at[0], rs_recv.at[0])
        rs0.start()
        ag1.wait_recv()

        ag2 = remote(chunks.at[1], chunks.at[2], ag_send.at[2], ag_recv.at[2])
        ag2.start()
        compute_partial(
            lambda rows: chunks[1, rows, :],
            lambda rows, v: acc.__setitem__((1, rows, slice(None)), v),
            lambda rows: acc[1, rows, :],
        )
        rs0.wait_recv()
        add_into_acc(1, 0)
        rs1 = remote(acc.at[1], rsrecv.at[1], rs_send.at[1], rs_recv.at[1])
        rs1.start()
        ag2.wait_recv()

        rs0.wait_send()
        compute_partial(
            lambda rows: chunks[2, rows, :],
            lambda rows, v: acc.__setitem__((0, rows, slice(None)), v),
            lambda rows: acc[0, rows, :],
        )
        rs1.wait_recv()
        add_into_acc(0, 1)
        rs2 = remote(acc.at[0], rsrecv.at[2], rs_send.at[2], rs_recv.at[2])
        rs2.start()

        rs2.wait_recv()

        def final_tile(ti, _):
            rows = pl.ds(ti * BT, BT)
            out_ref[rows, :] = rsrecv[2, rows, :] + p_own[rows, :]
            return 0

        lax.fori_loop(0, n_tiles, final_tile, 0, unroll=False)

        ag0.wait_send()
        ag1.wait_send()
        ag2.wait_send()
        rs1.wait_send()
        rs2.wait_send()

    return pl.pallas_call(
        body,
        out_shape=jax.ShapeDtypeStruct((t_loc, d), jnp.bfloat16),
        in_specs=[
            pl.BlockSpec(memory_space=pltpu.VMEM),
            pl.BlockSpec(memory_space=pltpu.MemorySpace.HBM),
            pl.BlockSpec(memory_space=pltpu.MemorySpace.HBM),
        ],
        out_specs=pl.BlockSpec(memory_space=pltpu.VMEM),
        scratch_shapes=[
            pltpu.VMEM((N_HOPS, t_loc, wcols), jnp.bfloat16),
            pltpu.VMEM((d, f), jnp.bfloat16),
            pltpu.VMEM((f, d), jnp.bfloat16),
            pltpu.VMEM((t_loc, d), jnp.bfloat16),
            pltpu.VMEM((2, t_loc, d), jnp.bfloat16),
            pltpu.VMEM((N_HOPS, t_loc, d), jnp.bfloat16),
            pltpu.SemaphoreType.DMA((N_HOPS,)),
            pltpu.SemaphoreType.DMA((N_HOPS,)),
            pltpu.SemaphoreType.DMA((N_HOPS,)),
            pltpu.SemaphoreType.DMA((N_HOPS,)),
            pltpu.SemaphoreType.DMA((2,)),
        ],
        compiler_params=pltpu.CompilerParams(
            vmem_limit_bytes=60 * 1024 * 1024,
        ),
    )(aug, w1, w2)


def kernel(x, assign, W1, W2):
    tloc, d = x.shape
    xb = x.astype(jnp.bfloat16)
    ab = jnp.broadcast_to(
        assign.astype(jnp.bfloat16)[:, None], (tloc, ASSIGN_PAD)
    )
    aug = jnp.concatenate([xb, ab], axis=1)
    out16 = _fused(aug, W1.astype(jnp.bfloat16), W2.astype(jnp.bfloat16))
    return out16.astype(jnp.float32)


# device time: 360741 ns/iter; 1.8654x vs baseline; 1.3290x over previous
import jax
import jax.numpy as jnp
from jax import lax
from jax.experimental import pallas as pl
from jax.experimental.pallas import tpu as pltpu

Z = 4
N_HOPS = Z - 1
ASSIGN_PAD = 128
BT = 256
CAP = 256


def _fused(aug, w1, w2):
    t_loc, wcols = aug.shape
    e_loc, d, f = w1.shape
    n_tiles = t_loc // BT
    bf16 = jnp.bfloat16

    def body(
        aug_ref, w1_hbm, w2_hbm, out_ref,
        chunks, wb1, wb2, p_own, acc, rsrecv, ranks_scr, masks_scr,
        ag_send, ag_recv, rs_send, rs_recv, w1sems, w2sem,
    ):
        mx = lax.axis_index("x")
        my = lax.axis_index("y")
        mz = lax.axis_index("z")
        right = (mz + 1) % Z
        dev = (mx, my, right)

        def remote(src, dst, ssem, rsem):
            return pltpu.make_async_remote_copy(
                src_ref=src, dst_ref=dst, send_sem=ssem, recv_sem=rsem,
                device_id=dev, device_id_type=pl.DeviceIdType.MESH,
            )

        def rd(ref, lead, rows, cols):
            if lead is None:
                return ref[rows, cols]
            return ref[lead, rows, cols]

        def wr(ref, lead, rows, val):
            if lead is None:
                ref[rows, :] = val
            else:
                ref[lead, rows, :] = val

        def compute_partial(src_ref, src_lead, dst_ref, dst_lead):
            allr = pl.ds(0, t_loc)
            e_iota = lax.broadcasted_iota(jnp.int32, (1, e_loc), 1)
            e_vals = (mz * e_loc + e_iota).astype(bf16)
            col_bt = lax.broadcasted_iota(
                jnp.int32, (BT, CAP), 1).astype(bf16)

            def mask_tile(ti, _):
                rows = pl.ds(ti * BT, BT)
                a_t = rd(src_ref, src_lead, rows, pl.ds(d, 1))
                masks_scr[rows, :] = (a_t == e_vals).astype(bf16)
                return 0

            lax.fori_loop(0, n_tiles, mask_tile, 0)

            def rank_tile(ti, _):
                masks_all = masks_scr[allr, :]
                r_iota = lax.broadcasted_iota(
                    jnp.int32, (BT, t_loc), 0) + ti * BT
                c_iota = lax.broadcasted_iota(jnp.int32, (BT, t_loc), 1)
                lt = (c_iota < r_iota).astype(bf16)
                rk = lax.dot_general(
                    lt, masks_all, (((1,), (0,)), ((), ())),
                    preferred_element_type=jnp.float32,
                )
                ranks_scr[pl.ds(ti * BT, BT), :] = rk.astype(bf16)
                return 0

            lax.fori_loop(0, n_tiles, rank_tile, 0)

            w1dma = [
                pltpu.make_async_copy(w1_hbm.at[e], wb1, w1sems.at[0])
                for e in range(e_loc)
            ]
            w2dma = [
                pltpu.make_async_copy(w2_hbm.at[e], wb2, w2sem)
                for e in range(e_loc)
            ]
            w1dma[0].start()
            w2dma[0].start()

            def disp_tile(e, ti):
                rows = pl.ds(ti * BT, BT)
                rk_t = ranks_scr[rows, pl.ds(e, 1)]
                mk_t = masks_scr[rows, pl.ds(e, 1)]
                return (
                    (rk_t == col_bt) & (mk_t > 0)
                ).astype(bf16)

            for e in range(e_loc):
                w1dma[e].wait()
                w2dma[e].wait()

                def disp_acc(ti, xc32, e=e):
                    rows = pl.ds(ti * BT, BT)
                    x_t = rd(src_ref, src_lead, rows, pl.ds(0, d))
                    return xc32 + lax.dot_general(
                        disp_tile(e, ti), x_t, (((0,), (0,)), ((), ())),
                        preferred_element_type=jnp.float32,
                    )

                xc32 = lax.fori_loop(
                    0, n_tiles, disp_acc, jnp.zeros((CAP, d), jnp.float32))
                xc = xc32.astype(bf16)
                h = jnp.dot(xc, wb1[...], preferred_element_type=jnp.float32)
                h = jnp.maximum(h, 0.0).astype(bf16)
                if e + 1 < e_loc:
                    w1dma[e + 1].start()
                yc = jnp.dot(
                    h, wb2[...], preferred_element_type=jnp.float32
                ).astype(bf16)
                if e + 1 < e_loc:
                    w2dma[e + 1].start()

                def combine_tile(ti, _, e=e, yc=yc):
                    rows = pl.ds(ti * BT, BT)
                    contrib = jnp.dot(
                        disp_tile(e, ti), yc,
                        preferred_element_type=jnp.float32,
                    ).astype(bf16)
                    if e == 0:
                        wr(dst_ref, dst_lead, rows, contrib)
                    else:
                        wr(dst_ref, dst_lead, rows,
                           rd(dst_ref, dst_lead, rows, slice(None)) + contrib)
                    return 0

                lax.fori_loop(0, n_tiles, combine_tile, 0)

        def add_into_acc(a_slot, r_slot):
            def tile(ti, _):
                rows = pl.ds(ti * BT, BT)
                acc[a_slot, rows, :] = (
                    acc[a_slot, rows, :] + rsrecv[r_slot, rows, :]
                )
                return 0

            lax.fori_loop(0, n_tiles, tile, 0)

        ag0 = remote(aug_ref, chunks.at[0], ag_send.at[0], ag_recv.at[0])
        ag0.start()
        compute_partial(aug_ref, None, p_own, None)
        ag0.wait_recv()

        ag1 = remote(chunks.at[0], chunks.at[1], ag_send.at[1], ag_recv.at[1])
        ag1.start()
        compute_partial(chunks, 0, acc, 0)
        rs0 = remote(acc.at[0], rsrecv.at[0], rs_send.at[0], rs_recv.at[0])
        rs0.start()
        ag1.wait_recv()

        ag2 = remote(chunks.at[1], chunks.at[2], ag_send.at[2], ag_recv.at[2])
        ag2.start()
        compute_partial(chunks, 1, acc, 1)
        rs0.wait_recv()
        add_into_acc(1, 0)
        rs1 = remote(acc.at[1], rsrecv.at[1], rs_send.at[1], rs_recv.at[1])
        rs1.start()
        ag2.wait_recv()

        rs0.wait_send()
        compute_partial(chunks, 2, acc, 0)
        rs1.wait_recv()
        add_into_acc(0, 1)
        rs2 = remote(acc.at[0], rsrecv.at[0], rs_send.at[2], rs_recv.at[2])
        rs2.start()

        rs2.wait_recv()

        def out_tile(ti, _):
            rows = pl.ds(ti * BT, BT)
            out_ref[rows, :] = rsrecv[0, rows, :] + p_own[rows, :]
            return 0

        lax.fori_loop(0, n_tiles, out_tile, 0)

        ag0.wait_send()
        ag1.wait_send()
        ag2.wait_send()
        rs1.wait_send()
        rs2.wait_send()

    return pl.pallas_call(
        body,
        out_shape=jax.ShapeDtypeStruct((t_loc, d), jnp.bfloat16),
        in_specs=[
            pl.BlockSpec(memory_space=pltpu.VMEM),
            pl.BlockSpec(memory_space=pltpu.MemorySpace.HBM),
            pl.BlockSpec(memory_space=pltpu.MemorySpace.HBM),
        ],
        out_specs=pl.BlockSpec(memory_space=pltpu.VMEM),
        scratch_shapes=[
            pltpu.VMEM((N_HOPS, t_loc, wcols), jnp.bfloat16),
            pltpu.VMEM((d, f), jnp.bfloat16),
            pltpu.VMEM((f, d), jnp.bfloat16),
            pltpu.VMEM((t_loc, d), jnp.bfloat16),
            pltpu.VMEM((2, t_loc, d), jnp.bfloat16),
            pltpu.VMEM((2, t_loc, d), jnp.bfloat16),
            pltpu.VMEM((t_loc, e_loc), jnp.bfloat16),
            pltpu.VMEM((t_loc, e_loc), jnp.bfloat16),
            pltpu.SemaphoreType.DMA((N_HOPS,)),
            pltpu.SemaphoreType.DMA((N_HOPS,)),
            pltpu.SemaphoreType.DMA((N_HOPS,)),
            pltpu.SemaphoreType.DMA((N_HOPS,)),
            pltpu.SemaphoreType.DMA((2,)),
            pltpu.SemaphoreType.DMA,
        ],
        compiler_params=pltpu.CompilerParams(
            vmem_limit_bytes=60 * 1024 * 1024,
        ),
    )(aug, w1, w2)


def kernel(x, assign, W1, W2):
    tloc, d = x.shape
    xb = x.astype(jnp.bfloat16)
    ab = jnp.broadcast_to(
        assign.astype(jnp.bfloat16)[:, None], (tloc, ASSIGN_PAD)
    )
    aug = jnp.concatenate([xb, ab], axis=1)
    out16 = _fused(aug, W1.astype(jnp.bfloat16), W2.astype(jnp.bfloat16))
    return out16.astype(jnp.float32)


# device time: 292071 ns/iter; 2.3040x vs baseline; 1.2351x over previous
import jax
import jax.numpy as jnp
from jax import lax
from jax.experimental import pallas as pl
from jax.experimental.pallas import tpu as pltpu

Z = 4
BT = 256
CAP = 256
N_EXP = 16


def _a2a(x, a_col, w1, w2):
    t_loc, d = x.shape
    e_loc, _, f = w1.shape
    n_tiles = t_loc // BT
    rows_buf = e_loc * CAP
    bf16 = jnp.bfloat16

    def body(
        x_ref, a_ref, w1_hbm, w2_hbm, out_ref,
        masks_scr, ranks_scr, sendb, fwdrecv, ownxc, ycown, ycret, retrecv,
        wb1, wb2,
        fwd_send, fwd_recv, ret_send, ret_recv, w1sems, w2sems,
    ):
        mx = lax.axis_index("x")
        my = lax.axis_index("y")
        mz = lax.axis_index("z")

        def remote(src, dst, ssem, rsem, dz):
            return pltpu.make_async_remote_copy(
                src_ref=src, dst_ref=dst, send_sem=ssem, recv_sem=rsem,
                device_id=(mx, my, dz), device_id_type=pl.DeviceIdType.MESH,
            )

        c_iota = lax.broadcasted_iota(jnp.int32, (1, N_EXP), 1)
        e_vals = (
            ((mz + c_iota // e_loc) % Z) * e_loc + c_iota % e_loc
        ).astype(bf16)
        col_bt = lax.broadcasted_iota(
            jnp.int32, (BT, CAP), 1).astype(bf16)

        def mask_tile(ti, _):
            rows = pl.ds(ti * BT, BT)
            masks_scr[rows, :] = (a_ref[rows, :] == e_vals).astype(bf16)
            return 0

        lax.fori_loop(0, n_tiles, mask_tile, 0)

        def rank_tile(ti, _):
            masks_all = masks_scr[...]
            r_iota = lax.broadcasted_iota(jnp.int32, (BT, t_loc), 0) + ti * BT
            cc = lax.broadcasted_iota(jnp.int32, (BT, t_loc), 1)
            lt = (cc < r_iota).astype(bf16)
            rk = lax.dot_general(
                lt, masks_all, (((1,), (0,)), ((), ())),
                preferred_element_type=jnp.float32,
            )
            ranks_scr[pl.ds(ti * BT, BT), :] = rk.astype(bf16)
            return 0

        lax.fori_loop(0, n_tiles, rank_tile, 0)

        def disp_tile(c, ti):
            rows = pl.ds(ti * BT, BT)
            rk_t = ranks_scr[rows, pl.ds(c, 1)]
            mk_t = masks_scr[rows, pl.ds(c, 1)]
            return ((rk_t == col_bt) & (mk_t > 0)).astype(bf16)

        def build_bucket(m, dst_ref, dst_lead):
            for e in range(e_loc):
                c = m * e_loc + e

                def acc_tile(ti, xc32, c=c):
                    rows = pl.ds(ti * BT, BT)
                    return xc32 + lax.dot_general(
                        disp_tile(c, ti), x_ref[rows, :],
                        (((0,), (0,)), ((), ())),
                        preferred_element_type=jnp.float32,
                    )

                xc = lax.fori_loop(
                    0, n_tiles, acc_tile, jnp.zeros((CAP, d), jnp.float32))
                erows = pl.ds(e * CAP, CAP)
                if dst_lead is None:
                    dst_ref[erows, :] = xc.astype(bf16)
                else:
                    dst_ref[dst_lead, erows, :] = xc.astype(bf16)

        def ffn(src_ref, src_lead, dst_ref, dst_lead):
            w1dma = [
                pltpu.make_async_copy(
                    w1_hbm.at[e], wb1.at[e % 2], w1sems.at[e % 2])
                for e in range(e_loc)
            ]
            w2dma = [
                pltpu.make_async_copy(
                    w2_hbm.at[e], wb2.at[e % 2], w2sems.at[e % 2])
                for e in range(e_loc)
            ]
            w1dma[0].start()
            w2dma[0].start()
            for e in range(e_loc):
                if e + 1 < e_loc:
                    w1dma[e + 1].start()
                    w2dma[e + 1].start()
                w1dma[e].wait()
                w2dma[e].wait()
                erows = pl.ds(e * CAP, CAP)
                if src_lead is None:
                    xc = src_ref[erows, :]
                else:
                    xc = src_ref[src_lead, erows, :]
                h = jnp.dot(
                    xc, wb1[e % 2], preferred_element_type=jnp.float32)
                h = jnp.maximum(h, 0.0).astype(bf16)
                yc = jnp.dot(
                    h, wb2[e % 2], preferred_element_type=jnp.float32)
                if dst_lead is None:
                    dst_ref[erows, :] = yc.astype(bf16)
                else:
                    dst_ref[dst_lead, erows, :] = yc.astype(bf16)

        fwd = []
        for m in (1, 2, 3):
            build_bucket(m, sendb, m - 1)
            r = remote(
                sendb.at[m - 1], fwdrecv.at[m - 1],
                fwd_send.at[m - 1], fwd_recv.at[m - 1], (mz + m) % Z,
            )
            r.start()
            fwd.append(r)

        build_bucket(0, ownxc, None)
        ffn(ownxc, None, ycown, None)

        ret = []
        for m in (1, 2, 3):
            fwd[m - 1].wait_recv()
            ffn(fwdrecv, m - 1, ycret, m - 1)
            r = remote(
                ycret.at[m - 1], retrecv.at[m - 1],
                ret_send.at[m - 1], ret_recv.at[m - 1], (mz - m) % Z,
            )
            r.start()
            ret.append(r)

        for r in ret:
            r.wait_recv()

        def out_tile(ti, _):
            rows = pl.ds(ti * BT, BT)
            acc = jnp.zeros((BT, d), jnp.float32)
            for m in range(Z):
                for e in range(e_loc):
                    c = m * e_loc + e
                    erows = pl.ds(e * CAP, CAP)
                    if m == 0:
                        yc = ycown[erows, :]
                    else:
                        yc = retrecv[m - 1, erows, :]
                    acc = acc + jnp.dot(
                        disp_tile(c, ti), yc,
                        preferred_element_type=jnp.float32,
                    )
            out_ref[rows, :] = acc.astype(bf16)
            return 0

        lax.fori_loop(0, n_tiles, out_tile, 0)

        for r in fwd:
            r.wait_send()
        for r in ret:
            r.wait_send()

    return pl.pallas_call(
        body,
        out_shape=jax.ShapeDtypeStruct((t_loc, d), jnp.bfloat16),
        in_specs=[
            pl.BlockSpec(memory_space=pltpu.VMEM),
            pl.BlockSpec(memory_space=pltpu.VMEM),
            pl.BlockSpec(memory_space=pltpu.MemorySpace.HBM),
            pl.BlockSpec(memory_space=pltpu.MemorySpace.HBM),
        ],
        out_specs=pl.BlockSpec(memory_space=pltpu.VMEM),
        scratch_shapes=[
            pltpu.VMEM((t_loc, N_EXP), jnp.bfloat16),
            pltpu.VMEM((t_loc, N_EXP), jnp.bfloat16),
            pltpu.VMEM((Z - 1, rows_buf, d), jnp.bfloat16),
            pltpu.VMEM((Z - 1, rows_buf, d), jnp.bfloat16),
            pltpu.VMEM((rows_buf, d), jnp.bfloat16),
            pltpu.VMEM((rows_buf, d), jnp.bfloat16),
            pltpu.VMEM((Z - 1, rows_buf, d), jnp.bfloat16),
            pltpu.VMEM((Z - 1, rows_buf, d), jnp.bfloat16),
            pltpu.VMEM((2, d, f), jnp.bfloat16),
            pltpu.VMEM((2, f, d), jnp.bfloat16),
            pltpu.SemaphoreType.DMA((Z - 1,)),
            pltpu.SemaphoreType.DMA((Z - 1,)),
            pltpu.SemaphoreType.DMA((Z - 1,)),
            pltpu.SemaphoreType.DMA((Z - 1,)),
            pltpu.SemaphoreType.DMA((2,)),
            pltpu.SemaphoreType.DMA((2,)),
        ],
        compiler_params=pltpu.CompilerParams(
            vmem_limit_bytes=60 * 1024 * 1024,
        ),
    )(x, a_col, w1, w2)


def kernel(x, assign, W1, W2):
    xb = x.astype(jnp.bfloat16)
    ab = assign.astype(jnp.bfloat16)[:, None]
    out16 = _a2a(xb, ab, W1.astype(jnp.bfloat16), W2.astype(jnp.bfloat16))
    return out16.astype(jnp.float32)
